# baseline (speedup 1.0000x reference)
import dataclasses

import jax
import jax.numpy as jnp
from jax import lax
from jax.experimental import pallas as pl
from jax.experimental.pallas import tpu as pltpu
from jax.experimental.pallas import tpu_sc as plsc

N = 1048576
CH = 8
DIMS = (64, 64, 64, 32)
STRIDES = (DIMS[1] * DIMS[2] * DIMS[3], DIMS[2] * DIMS[3], DIMS[3], 1)
NW = 32
PPW = N // NW
B = 128
NBLK = PPW // B
NSG = B // 16
NROW = 16 * B
NSTREAM = NROW // 128


def _body(xs_hbm, data_hbm, out_hbm, xsb, idxb, wbuf, gbuf, outb, sem):
    wid = lax.axis_index("s") * 2 + lax.axis_index("c")
    base_pt = wid * PPW

    iota16 = lax.iota(jnp.int32, 16)
    cconst = [jnp.full((16,), c, jnp.int32) for c in range(CH)]

    @pl.loop(0, NBLK)
    def _(blk):
        start = base_pt + blk * B
        pltpu.sync_copy(xs_hbm.at[pl.ds(start, B)], xsb)

        @pl.loop(0, NSG)
        def _(sg):
            p16 = iota16 + sg * 16
            lo = []
            hi = []
            f = []
            for d in range(4):
                cd = plsc.load_gather(xsb, [p16, jnp.full((16,), d, jnp.int32)])
                half = 0.5 * DIMS[d]
                sd = cd * half + half
                idx_i = sd.astype(jnp.int32)
                f.append(sd - idx_i.astype(jnp.float32))
                lod = jnp.minimum(jnp.maximum(idx_i, 0), DIMS[d] - 1)
                lo.append(lod)
                hi.append(jnp.minimum(lod + 1, DIMS[d] - 1))
            px = [lo[0] * STRIDES[0], hi[0] * STRIDES[0]]
            py = [lo[1] * STRIDES[1], hi[1] * STRIDES[1]]
            pz = [lo[2] * STRIDES[2], hi[2] * STRIDES[2]]
            pt = [lo[3], hi[3]]
            wx = [1.0 - f[0], f[0]]
            wy = [1.0 - f[1], f[1]]
            wz = [1.0 - f[2], f[2]]
            wt = [1.0 - f[3], f[3]]
            pxy = {}
            wxy = {}
            for bx in range(2):
                for by in range(2):
                    pxy[(bx, by)] = px[bx] + py[by]
                    wxy[(bx, by)] = wx[bx] * wy[by]
            pxyz = {}
            wxyz = {}
            for bx in range(2):
                for by in range(2):
                    for bz in range(2):
                        pxyz[(bx, by, bz)] = pxy[(bx, by)] + pz[bz]
                        wxyz[(bx, by, bz)] = wxy[(bx, by)] * wz[bz]
            for b in range(16):
                bx, by, bz, bt = b & 1, (b >> 1) & 1, (b >> 2) & 1, (b >> 3) & 1
                ofs = sg * 256 + b * 16
                idxb[pl.ds(ofs, 16)] = pxyz[(bx, by, bz)] + pt[bt]
                wbuf[pl.ds(ofs, 16)] = wxyz[(bx, by, bz)] * wt[bt]

        copies = []
        for j in range(NSTREAM):
            copies.append(
                pltpu.async_copy(
                    data_hbm.at[idxb.at[pl.ds(j * 128, 128)]],
                    gbuf.at[pl.ds(j * 128, 128)],
                    sem,
                )
            )
        for c in copies:
            c.wait()

        @pl.loop(0, NSG)
        def _(sg):
            base = sg * 256
            acc = [None] * CH
            for r in range(16):
                w_r = wbuf[pl.ds(base + r * 16, 16)]
                kvec = iota16 + (base + r * 16)
                for c in range(CH):
                    col = plsc.load_gather(gbuf, [kvec, cconst[c]])
                    term = w_r * col
                    acc[c] = term if acc[c] is None else acc[c] + term
            p16 = iota16 + sg * 16
            for c in range(CH):
                plsc.store_scatter(outb, [p16, cconst[c]], acc[c])

        pltpu.sync_copy(outb, out_hbm.at[pl.ds(start, B)])


def kernel(xs, data):
    xs = xs.reshape(-1, 4)
    data2 = data.reshape(-1, CH)
    mesh = plsc.VectorSubcoreMesh(core_axis_name="c", subcore_axis_name="s")
    cp = pltpu.CompilerParams()
    if "needs_layout_passes" in pltpu.CompilerParams.__dataclass_fields__:
        cp = dataclasses.replace(cp, needs_layout_passes=False)
    if "use_tc_tiling_on_sc" in pltpu.CompilerParams.__dataclass_fields__:
        cp = dataclasses.replace(cp, use_tc_tiling_on_sc=False)
    k = pl.kernel(
        _body,
        out_type=jax.ShapeDtypeStruct((N, CH), jnp.float32),
        mesh=mesh,
        scratch_types=[
            pltpu.VMEM((B, 4), jnp.float32),
            pltpu.VMEM((NROW,), jnp.int32),
            pltpu.VMEM((NROW,), jnp.float32),
            pltpu.VMEM((NROW, CH), jnp.float32),
            pltpu.VMEM((B, CH), jnp.float32),
            pltpu.SemaphoreType.DMA,
        ],
        compiler_params=cp,
    )
    return k(xs, data2)


# speedup vs baseline: 1.0549x; 1.0549x over previous
import dataclasses

import jax
import jax.numpy as jnp
from jax import lax
from jax.experimental import pallas as pl
from jax.experimental.pallas import tpu as pltpu
from jax.experimental.pallas import tpu_sc as plsc

N = 1048576
CH = 8
DIMS = (64, 64, 64, 32)
STRIDES = (DIMS[1] * DIMS[2] * DIMS[3], DIMS[2] * DIMS[3], DIMS[3], 1)
NW = 32
PPW = N // NW
B = 128
NBLK = PPW // B
NSG = B // 16
NROW = 16 * B
NSTREAM = NROW // 128


def _body(xs_hbm, data_hbm, out_hbm, xsb, idxb, wbuf, gbuf, outc, sem):
    wid = lax.axis_index("s") * 2 + lax.axis_index("c")
    base_pt = wid * PPW

    iota16 = lax.iota(jnp.int32, 16)
    cconst = [jnp.full((16,), c, jnp.int32) for c in range(CH)]

    @pl.loop(0, NBLK)
    def _(blk):
        start = base_pt + blk * B
        for d in range(4):
            pltpu.sync_copy(xs_hbm.at[d, pl.ds(start, B)], xsb.at[d])

        @pl.loop(0, NSG)
        def _(sg):
            lo = []
            hi = []
            f = []
            for d in range(4):
                cd = xsb[d, pl.ds(sg * 16, 16)]
                half = 0.5 * DIMS[d]
                sd = cd * half + half
                idx_i = sd.astype(jnp.int32)
                f.append(sd - idx_i.astype(jnp.float32))
                lod = jnp.minimum(jnp.maximum(idx_i, 0), DIMS[d] - 1)
                lo.append(lod)
                hi.append(jnp.minimum(lod + 1, DIMS[d] - 1))
            px = [lo[0] * STRIDES[0], hi[0] * STRIDES[0]]
            py = [lo[1] * STRIDES[1], hi[1] * STRIDES[1]]
            pz = [lo[2] * STRIDES[2], hi[2] * STRIDES[2]]
            pt = [lo[3], hi[3]]
            wx = [1.0 - f[0], f[0]]
            wy = [1.0 - f[1], f[1]]
            wz = [1.0 - f[2], f[2]]
            wt = [1.0 - f[3], f[3]]
            pxy = {}
            wxy = {}
            for bx in range(2):
                for by in range(2):
                    pxy[(bx, by)] = px[bx] + py[by]
                    wxy[(bx, by)] = wx[bx] * wy[by]
            pxyz = {}
            wxyz = {}
            for bx in range(2):
                for by in range(2):
                    for bz in range(2):
                        pxyz[(bx, by, bz)] = pxy[(bx, by)] + pz[bz]
                        wxyz[(bx, by, bz)] = wxy[(bx, by)] * wz[bz]
            for b in range(16):
                bx, by, bz, bt = b & 1, (b >> 1) & 1, (b >> 2) & 1, (b >> 3) & 1
                ofs = sg * 256 + b * 16
                idxb[pl.ds(ofs, 16)] = pxyz[(bx, by, bz)] + pt[bt]
                wbuf[pl.ds(ofs, 16)] = wxyz[(bx, by, bz)] * wt[bt]

        copies = []
        for j in range(NSTREAM):
            copies.append(
                pltpu.async_copy(
                    data_hbm.at[idxb.at[pl.ds(j * 128, 128)]],
                    gbuf.at[pl.ds(j * 128, 128)],
                    sem,
                )
            )
        for c in copies:
            c.wait()

        @pl.loop(0, NSG)
        def _(sg):
            base = sg * 256
            acc = [None] * CH
            for r in range(16):
                w_r = wbuf[pl.ds(base + r * 16, 16)]
                kvec = iota16 + (base + r * 16)
                for c in range(CH):
                    col = plsc.load_gather(gbuf, [kvec, cconst[c]])
                    term = w_r * col
                    acc[c] = term if acc[c] is None else acc[c] + term
            for c in range(CH):
                outc[c, pl.ds(sg * 16, 16)] = acc[c]

        for c in range(CH):
            pltpu.sync_copy(outc.at[c], out_hbm.at[c, pl.ds(start, B)])


def kernel(xs, data):
    xs_t = xs.T
    data2 = data.reshape(-1).reshape(-1, CH)
    mesh = plsc.VectorSubcoreMesh(core_axis_name="c", subcore_axis_name="s")
    cp = pltpu.CompilerParams()
    if "needs_layout_passes" in pltpu.CompilerParams.__dataclass_fields__:
        cp = dataclasses.replace(cp, needs_layout_passes=False)
    if "use_tc_tiling_on_sc" in pltpu.CompilerParams.__dataclass_fields__:
        cp = dataclasses.replace(cp, use_tc_tiling_on_sc=False)
    k = pl.kernel(
        _body,
        out_type=jax.ShapeDtypeStruct((CH, N), jnp.float32),
        mesh=mesh,
        scratch_types=[
            pltpu.VMEM((4, B), jnp.float32),
            pltpu.VMEM((NROW,), jnp.int32),
            pltpu.VMEM((NROW,), jnp.float32),
            pltpu.VMEM((NROW, CH), jnp.float32),
            pltpu.VMEM((CH, B), jnp.float32),
            pltpu.SemaphoreType.DMA,
        ],
        compiler_params=cp,
    )
    out = k(xs_t, data2)
    return out.T
